# baseline (speedup 1.0000x reference)
import functools

import jax
import jax.numpy as jnp
from jax import lax
from jax.experimental import pallas as pl
from jax.experimental.pallas import tpu as pltpu
from jax.experimental.pallas import tpu_sc as plsc

_ROWS = 128
_N = 32768
_L = 16
_NV = _N // _L
_U = 8
_BISECT_ITERS = 22

_info = plsc.get_sparse_core_info()
_NC = _info.num_cores
_NS = _info.num_subcores
_NW = _NC * _NS
_ROWS_PER_W = _ROWS // _NW


def _row_sparsemax(row_v, cand_v):
    sm = row_v[pl.ds(0, _L)]
    for i in range(1, 64):
        sm = jnp.maximum(sm, row_v[pl.ds(i * (_N // 64), _L)])
    t0 = jnp.max(sm) - 1.0
    t0_v = jnp.zeros((_L,), jnp.float32) + t0

    _CU = _U
    REG = _N // _CU
    iota = lax.iota(jnp.int32, _L)
    sixteen = jnp.full((_L,), _L, jnp.int32)
    zero_i = jnp.zeros((_L,), jnp.int32)

    def cp_body(i, idxs):
        base = i * (_CU * _L)
        new = []
        for j in range(_CU):
            v = row_v[pl.ds(base + j * _L, _L)]
            msk = v > t0_v
            plsc.store_scatter(cand_v, [idxs[j]], v, mask=msk)
            new.append(idxs[j] + jnp.where(msk, sixteen, zero_i))
        return tuple(new)

    idxs0 = tuple(iota + j * REG for j in range(_CU))
    idxs = plsc.parallel_loop(0, _NV // _CU, 1, unroll=1, carry=idxs0)(cp_body)
    lo0_v = t0_v
    depth = idxs[0] - idxs0[0]
    for j in range(1, _CU):
        depth = jnp.maximum(depth, idxs[j] - idxs0[j])
    nb = (jnp.max(depth) >> 4) + 1
    limits = [jnp.minimum(nb * _L, REG) + idxs0[j] for j in range(_CU)]

    def fill_body(_, idxs):
        new = []
        for j in range(_CU):
            msk = idxs[j] < limits[j]
            plsc.store_scatter(cand_v, [idxs[j]], lo0_v, mask=msk)
            new.append(idxs[j] + jnp.where(msk, sixteen, zero_i))
        return tuple(new)

    plsc.parallel_loop(0, nb, 1, unroll=1, carry=tuple(idxs))(fill_body)

    def m_body(i, acc):
        for j in range(_CU):
            acc = jnp.maximum(acc, cand_v[pl.ds(j * REG + i * _L, _L)])
        return acc

    m = jnp.max(lax.fori_loop(0, nb, m_body, t0_v))
    lo0 = jnp.maximum(t0, m - 1.0)

    def f_eval(t):
        def body(i, acc):
            for j in range(_CU):
                v = cand_v[pl.ds(j * REG + i * _L, _L)]
                acc = acc + jnp.maximum(v - t, 0.0)
            return acc

        return jnp.sum(lax.fori_loop(0, nb, body, jnp.zeros((_L,), jnp.float32)))

    def bis_body(_, carry):
        lo, hi = carry
        mid = 0.5 * (lo + hi)
        pos = f_eval(mid) > 1.0
        return jnp.where(pos, mid, lo), jnp.where(pos, hi, mid)

    lo, hi = lax.fori_loop(0, _BISECT_ITERS, bis_body, (lo0, m))
    tau_b = 0.5 * (lo + hi)

    def rf_body(i, carry):
        s, k = carry
        for j in range(_CU):
            v = cand_v[pl.ds(j * REG + i * _L, _L)]
            msk = v > tau_b
            s = s + jnp.where(msk, v, 0.0)
            k = k + jnp.where(msk, 1.0, 0.0)
        return s, k

    z16 = jnp.zeros((_L,), jnp.float32)
    s_v, k_v = lax.fori_loop(0, nb, rf_body, (z16, z16))
    s, k = jnp.sum(s_v), jnp.sum(k_v)
    ratio_v = (z16 + (s - 1.0)) / (z16 + jnp.maximum(k, 1.0))
    tau_v = jnp.where(z16 + k > 0.5, ratio_v, z16 + tau_b)

    def out_body(i, carry):
        base = i * (_U * _L)
        for j in range(_U):
            sl = pl.ds(base + j * _L, _L)
            row_v[sl] = jnp.maximum(row_v[sl] - tau_v, 0.0)
        return carry

    lax.fori_loop(0, _NV // _U, out_body, jnp.int32(0))


def _sc_sparsemax_body(x_hbm, out_hbm, row0_v, row1_v, cand_v, in_sems, out_sems):
    wid = lax.axis_index("s") * _NC + lax.axis_index("c")
    bufs = (row0_v, row1_v)

    def start_in(r, b):
        return pltpu.async_copy(
            x_hbm.at[wid * _ROWS_PER_W + r], bufs[b], in_sems.at[b]
        )

    def start_out(r, b):
        return pltpu.async_copy(
            bufs[b], out_hbm.at[wid * _ROWS_PER_W + r], out_sems.at[b]
        )

    in_dma = {0: start_in(0, 0)}
    out_dma = {}
    for r in range(_ROWS_PER_W):
        b = r % 2
        in_dma.pop(b).wait()
        if r + 1 < _ROWS_PER_W:
            nb_ = (r + 1) % 2
            if nb_ in out_dma:
                out_dma.pop(nb_).wait()
            in_dma[nb_] = start_in(r + 1, nb_)
        _row_sparsemax(bufs[b], cand_v)
        out_dma[b] = start_out(r, b)
    for b in sorted(out_dma):
        out_dma.pop(b).wait()


_sc_sparsemax = functools.partial(
    pl.kernel,
    out_type=jax.ShapeDtypeStruct((_ROWS, _N), jnp.float32),
    mesh=plsc.VectorSubcoreMesh(core_axis_name="c", subcore_axis_name="s"),
    compiler_params=pltpu.CompilerParams(needs_layout_passes=False),
    scratch_types=[
        pltpu.VMEM((_N,), jnp.float32),
        pltpu.VMEM((_N,), jnp.float32),
        pltpu.VMEM((_N + _L,), jnp.float32),
        pltpu.SemaphoreType.DMA((2,)),
        pltpu.SemaphoreType.DMA((2,)),
    ],
)(_sc_sparsemax_body)


def kernel(x):
    return _sc_sparsemax(x)


# speedup vs baseline: 1.1352x; 1.1352x over previous
import functools

import jax
import jax.numpy as jnp
from jax import lax
from jax.experimental import pallas as pl
from jax.experimental.pallas import tpu as pltpu
from jax.experimental.pallas import tpu_sc as plsc

_ROWS = 128
_N = 32768
_L = 16
_NV = _N // _L
_U = 8
_BISECT_ITERS = 22

_info = plsc.get_sparse_core_info()
_NC = _info.num_cores
_NS = _info.num_subcores
_NW = _NC * _NS
_ROWS_PER_W = _ROWS // _NW


def _row_sparsemax(row_v, cand_v):
    sm = row_v[pl.ds(0, _L)]
    for i in range(1, 64):
        sm = jnp.maximum(sm, row_v[pl.ds(i * (_N // 64), _L)])
    t0 = jnp.max(sm) - 1.0
    t0_v = jnp.zeros((_L,), jnp.float32) + t0

    _CU = _U
    REG = _N // _CU
    iota = lax.iota(jnp.int32, _L)
    sixteen = jnp.full((_L,), _L, jnp.int32)
    zero_i = jnp.zeros((_L,), jnp.int32)

    def cp_body(i, idxs):
        base = i * (_CU * _L)
        new = []
        for j in range(_CU):
            v = row_v[pl.ds(base + j * _L, _L)]
            msk = v > t0_v
            plsc.store_scatter(cand_v, [idxs[j]], v, mask=msk)
            new.append(idxs[j] + jnp.where(msk, sixteen, zero_i))
        return tuple(new)

    idxs0 = tuple(iota + j * REG for j in range(_CU))
    idxs = plsc.parallel_loop(0, _NV // _CU, 1, unroll=1, carry=idxs0)(cp_body)
    lo0_v = t0_v
    depth = idxs[0] - idxs0[0]
    for j in range(1, _CU):
        depth = jnp.maximum(depth, idxs[j] - idxs0[j])
    nb = (jnp.max(depth) >> 4) + 1
    limits = [jnp.minimum(nb * _L, REG) + idxs0[j] for j in range(_CU)]

    def fill_body(_, idxs):
        new = []
        for j in range(_CU):
            msk = idxs[j] < limits[j]
            plsc.store_scatter(cand_v, [idxs[j]], lo0_v, mask=msk)
            new.append(idxs[j] + jnp.where(msk, sixteen, zero_i))
        return tuple(new)

    plsc.parallel_loop(0, nb, 1, unroll=1, carry=tuple(idxs))(fill_body)

    def m_body(i, acc):
        for j in range(_CU):
            acc = jnp.maximum(acc, cand_v[pl.ds(j * REG + i * _L, _L)])
        return acc

    m = jnp.max(lax.fori_loop(0, nb, m_body, t0_v))
    lo0 = jnp.maximum(t0, m - 1.0)

    z16 = jnp.zeros((_L,), jnp.float32)

    def mset_eval(t):
        def body(i, c2):
            sacc, kacc = c2
            for j in range(_CU):
                v = cand_v[pl.ds(j * REG + i * _L, _L)]
                msk = v > t
                sacc = sacc + jnp.where(msk, v, 0.0)
                kacc = kacc + jnp.where(msk, 1.0, 0.0)
            return sacc, kacc

        s_v, k_v = lax.fori_loop(0, nb, body, (z16, z16))
        return jnp.sum(s_v), jnp.sum(k_v)

    def mi_body(_, carry):
        tau, _k = carry
        s, k = mset_eval(tau)
        tau_new_v = (z16 + (s - 1.0)) / (z16 + jnp.maximum(k, 1.0))
        return tau_new_v[0], k

    tau8, k8 = lax.fori_loop(0, 9, mi_body, (lo0, jnp.float32(0.0)))
    s9, k9 = mset_eval(tau8)
    tau9 = ((z16 + (s9 - 1.0)) / (z16 + jnp.maximum(k9, 1.0)))[0]
    converged = jnp.logical_and(k9 == k8, k9 > 0.5)

    def f_eval(t):
        def body(i, acc):
            for j in range(_CU):
                v = cand_v[pl.ds(j * REG + i * _L, _L)]
                acc = acc + jnp.maximum(v - t, 0.0)
            return acc

        return jnp.sum(lax.fori_loop(0, nb, body, z16))

    def _fallback():
        def bis_body(_, carry):
            lo, hi = carry
            mid = 0.5 * (lo + hi)
            pos = f_eval(mid) > 1.0
            return jnp.where(pos, mid, lo), jnp.where(pos, hi, mid)

        lo, hi = lax.fori_loop(0, _BISECT_ITERS, bis_body, (lo0, m))
        tau_b = 0.5 * (lo + hi)
        s, k = mset_eval(tau_b)
        ratio = ((z16 + (s - 1.0)) / (z16 + jnp.maximum(k, 1.0)))[0]
        return jnp.where(k > 0.5, ratio, tau_b)

    tau = lax.cond(converged, lambda: tau9, _fallback)
    tau_v = z16 + tau

    def out_body(i, carry):
        base = i * (_U * _L)
        for j in range(_U):
            sl = pl.ds(base + j * _L, _L)
            row_v[sl] = jnp.maximum(row_v[sl] - tau_v, 0.0)
        return carry

    lax.fori_loop(0, _NV // _U, out_body, jnp.int32(0))


def _sc_sparsemax_body(x_hbm, out_hbm, row0_v, row1_v, cand_v, in_sems, out_sems):
    wid = lax.axis_index("s") * _NC + lax.axis_index("c")
    bufs = (row0_v, row1_v)

    def start_in(r, b):
        return pltpu.async_copy(
            x_hbm.at[wid * _ROWS_PER_W + r], bufs[b], in_sems.at[b]
        )

    def start_out(r, b):
        return pltpu.async_copy(
            bufs[b], out_hbm.at[wid * _ROWS_PER_W + r], out_sems.at[b]
        )

    in_dma = {0: start_in(0, 0)}
    out_dma = {}
    for r in range(_ROWS_PER_W):
        b = r % 2
        in_dma.pop(b).wait()
        if r + 1 < _ROWS_PER_W:
            nb_ = (r + 1) % 2
            if nb_ in out_dma:
                out_dma.pop(nb_).wait()
            in_dma[nb_] = start_in(r + 1, nb_)
        _row_sparsemax(bufs[b], cand_v)
        out_dma[b] = start_out(r, b)
    for b in sorted(out_dma):
        out_dma.pop(b).wait()


_sc_sparsemax = functools.partial(
    pl.kernel,
    out_type=jax.ShapeDtypeStruct((_ROWS, _N), jnp.float32),
    mesh=plsc.VectorSubcoreMesh(core_axis_name="c", subcore_axis_name="s"),
    compiler_params=pltpu.CompilerParams(needs_layout_passes=False),
    scratch_types=[
        pltpu.VMEM((_N,), jnp.float32),
        pltpu.VMEM((_N,), jnp.float32),
        pltpu.VMEM((_N + _L,), jnp.float32),
        pltpu.SemaphoreType.DMA((2,)),
        pltpu.SemaphoreType.DMA((2,)),
    ],
)(_sc_sparsemax_body)


def kernel(x):
    return _sc_sparsemax(x)
